# Initial kernel scaffold; baseline (speedup 1.0000x reference)
import jax
import jax.numpy as jnp
from jax.experimental import pallas as pl


def kernel(x_customer, x_recipe, edge_index, edge_label_index, W1_l_c2r, b1_l_c2r, W1_r_c2r, W1_l_r2c, b1_l_r2c, W1_r_r2c, W2_l_c2r, b2_l_c2r, W2_r_c2r, W2_l_r2c, b2_l_r2c, W2_r_r2c, dec_W1, dec_b1, dec_W2, dec_b2):
    raise NotImplementedError("write your pallas kernel here")


# speedup vs baseline: 3.9862x; 3.9862x over previous
import functools

import jax
import jax.numpy as jnp
from jax import lax
from jax.experimental import pallas as pl
from jax.experimental.pallas import tpu as pltpu
from jax.experimental.pallas import tpu_sc as plsc

NC, NS, LANES = 2, 16, 16
NW = NC * NS
N = 10000
D = 128
E = 320000
NLBL = 65536

EPW = E // NW
ECH = 80
NECH = EPW // ECH

EPT = E // NS
NCCH = EPT // ECH

NPAD = 10240
HPT = NPAD // NS

LPW = NLBL // NW
LCH = 128
NLCH = LPW // LCH

ROWS_PT = 624
ROWS_REM = N - NS * ROWS_PT

_MESH = plsc.VectorSubcoreMesh(core_axis_name="c", subcore_axis_name="s")
_PREC = lax.Precision.HIGHEST


def _dot(a, b):
    return lax.dot_general(a, b, (((1,), (0,)), ((), ())),
                           precision=_PREC, preferred_element_type=jnp.float32)


def _counts_body(cat_hbm, ones_hbm, zeros_hbm, out_hbm, cnt_sh, idx_v, ones_v, sem):
    c = lax.axis_index("c")
    s = lax.axis_index("s")
    pltpu.sync_copy(zeros_hbm.at[pl.ds(s * HPT, HPT)], cnt_sh.at[pl.ds(s * HPT, HPT)])
    pltpu.sync_copy(ones_hbm, ones_v)
    plsc.subcore_barrier()

    base0 = c * E + s * EPT

    @pl.loop(0, NCCH)
    def _chunk(i):
        pltpu.sync_copy(cat_hbm.at[pl.ds(base0 + i * ECH, ECH)], idx_v)
        pltpu.sync_copy(ones_v, cnt_sh.at[idx_v], add=True)

    plsc.subcore_barrier()
    pltpu.sync_copy(cnt_sh.at[pl.ds(s * HPT, HPT)],
                    out_hbm.at[c, pl.ds(s * HPT, HPT)])


@functools.partial(
    pl.kernel,
    out_type=jax.ShapeDtypeStruct((NC, NPAD), jnp.float32),
    mesh=_MESH,
    scratch_types=[
        pltpu.VMEM_SHARED((NPAD,), jnp.float32),
        pltpu.VMEM((ECH,), jnp.int32),
        pltpu.VMEM((ECH,), jnp.float32),
        pltpu.SemaphoreType.DMA,
    ],
)
def _counts(cat_hbm, ones_hbm, zeros_hbm, out_hbm, cnt_sh, idx_v, ones_v, sem):
    _counts_body(cat_hbm, ones_hbm, zeros_hbm, out_hbm, cnt_sh, idx_v, ones_v, sem)


def _segsum_body(x_hbm, src_hbm, dst_hbm, zeros_hbm, out_hbm,
                 acc_sh, idx_s, idx_d, rows_v, sem):
    c = lax.axis_index("c")
    s = lax.axis_index("s")
    wid = c * NS + s
    r0 = s * ROWS_PT
    pltpu.sync_copy(zeros_hbm.at[pl.ds(r0, ROWS_PT)], acc_sh.at[pl.ds(r0, ROWS_PT)])

    @pl.when(s == 0)
    def _init_rem():
        pltpu.sync_copy(zeros_hbm.at[pl.ds(NS * ROWS_PT, ROWS_REM)],
                        acc_sh.at[pl.ds(NS * ROWS_PT, ROWS_REM)])

    plsc.subcore_barrier()

    base0 = wid * EPW

    @pl.loop(0, NECH)
    def _chunk(i):
        base = base0 + i * ECH
        pltpu.sync_copy(src_hbm.at[pl.ds(base, ECH)], idx_s)
        pltpu.sync_copy(dst_hbm.at[pl.ds(base, ECH)], idx_d)
        pltpu.async_copy(x_hbm.at[idx_s], rows_v, sem).wait()
        pltpu.sync_copy(rows_v, acc_sh.at[idx_d], add=True)

    plsc.subcore_barrier()
    pltpu.sync_copy(acc_sh.at[pl.ds(r0, ROWS_PT)],
                    out_hbm.at[c, pl.ds(r0, ROWS_PT)])

    @pl.when(s == 0)
    def _export_rem():
        pltpu.sync_copy(acc_sh.at[pl.ds(NS * ROWS_PT, ROWS_REM)],
                        out_hbm.at[c, pl.ds(NS * ROWS_PT, ROWS_REM)])


@functools.partial(
    pl.kernel,
    out_type=jax.ShapeDtypeStruct((NC, N, D), jnp.float32),
    mesh=_MESH,
    scratch_types=[
        pltpu.VMEM_SHARED((N, D), jnp.float32),
        pltpu.VMEM((ECH,), jnp.int32),
        pltpu.VMEM((ECH,), jnp.int32),
        pltpu.VMEM((ECH, D), jnp.float32),
        pltpu.SemaphoreType.DMA,
    ],
)
def _segment_sum(x_hbm, src_hbm, dst_hbm, zeros_hbm, out_hbm,
                 acc_sh, idx_s, idx_d, rows_v, sem):
    _segsum_body(x_hbm, src_hbm, dst_hbm, zeros_hbm, out_hbm,
                 acc_sh, idx_s, idx_d, rows_v, sem)


def _gather_pair_body(pc_hbm, pr_hbm, row_hbm, col_hbm, g_hbm,
                      ridx, cidx, buf_v, sem_a, sem_b):
    c = lax.axis_index("c")
    s = lax.axis_index("s")
    wid = c * NS + s
    base0 = wid * LPW

    @pl.loop(0, NLCH)
    def _chunk(i):
        base = base0 + i * LCH
        pltpu.sync_copy(row_hbm.at[pl.ds(base, LCH)], ridx)
        pltpu.sync_copy(col_hbm.at[pl.ds(base, LCH)], cidx)
        pltpu.async_copy(pc_hbm.at[ridx], buf_v, sem_a).wait()
        pltpu.async_copy(pr_hbm.at[cidx], buf_v, sem_b, add=True).wait()
        pltpu.sync_copy(buf_v, g_hbm.at[pl.ds(base, LCH)])


@functools.partial(
    pl.kernel,
    out_type=jax.ShapeDtypeStruct((NLBL, D), jnp.float32),
    mesh=_MESH,
    scratch_types=[
        pltpu.VMEM((LCH,), jnp.int32),
        pltpu.VMEM((LCH,), jnp.int32),
        pltpu.VMEM((LCH, D), jnp.float32),
        pltpu.SemaphoreType.DMA,
        pltpu.SemaphoreType.DMA,
    ],
)
def _gather_pair(pc_hbm, pr_hbm, row_hbm, col_hbm, g_hbm,
                 ridx, cidx, buf_v, sem_a, sem_b):
    _gather_pair_body(pc_hbm, pr_hbm, row_hbm, col_hbm, g_hbm,
                      ridx, cidx, buf_v, sem_a, sem_b)


BM = 2000


def _combine_relu_body(acc_ref, cnt_ref, x_ref, wl_ref, bl_ref, wr_ref, o_ref):
    inv = 1.0 / jnp.maximum(cnt_ref[...], 1.0)
    mean = (acc_ref[0] + acc_ref[1]) * inv
    h = _dot(mean, wl_ref[...]) + _dot(x_ref[...], wr_ref[...]) + bl_ref[...]
    o_ref[...] = jnp.maximum(h, 0.0)


def _combine_proj_body(acc_ref, cnt_ref, x_ref, wl_ref, bl_ref, wr_ref,
                       dw_ref, eb_ref, o_ref):
    inv = 1.0 / jnp.maximum(cnt_ref[...], 1.0)
    mean = (acc_ref[0] + acc_ref[1]) * inv
    dw = dw_ref[...]
    wld = _dot(wl_ref[...], dw)
    wrd = _dot(wr_ref[...], dw)
    bld = _dot(bl_ref[...], dw)
    o_ref[...] = _dot(mean, wld) + _dot(x_ref[...], wrd) + bld + eb_ref[...]


_acc_spec = pl.BlockSpec((NC, BM, D), lambda i: (0, i, 0))
_cnt_spec = pl.BlockSpec((BM, 1), lambda i: (i, 0))
_x_spec = pl.BlockSpec((BM, D), lambda i: (i, 0))
_w_spec = pl.BlockSpec((D, D), lambda i: (0, 0))
_b_spec = pl.BlockSpec((1, D), lambda i: (0, 0))

_combine_relu = pl.pallas_call(
    _combine_relu_body,
    grid=(N // BM,),
    in_specs=[_acc_spec, _cnt_spec, _x_spec, _w_spec, _b_spec, _w_spec],
    out_specs=_x_spec,
    out_shape=jax.ShapeDtypeStruct((N, D), jnp.float32),
)

_combine_proj = pl.pallas_call(
    _combine_proj_body,
    grid=(N // BM,),
    in_specs=[_acc_spec, _cnt_spec, _x_spec, _w_spec, _b_spec, _w_spec,
              _w_spec, _b_spec],
    out_specs=_x_spec,
    out_shape=jax.ShapeDtypeStruct((N, D), jnp.float32),
)


def _decode_body(g_ref, w2_ref, b2_ref, o_ref):
    o_ref[...] = _dot(jnp.maximum(g_ref[...], 0.0), w2_ref[...]) + b2_ref[...]


BL = 8192

_decode = pl.pallas_call(
    _decode_body,
    grid=(NLBL // BL,),
    in_specs=[pl.BlockSpec((BL, D), lambda i: (i, 0)),
              pl.BlockSpec((D, 1), lambda i: (0, 0)),
              pl.BlockSpec((1, 1), lambda i: (0, 0))],
    out_specs=pl.BlockSpec((BL, 1), lambda i: (i, 0)),
    out_shape=jax.ShapeDtypeStruct((NLBL, 1), jnp.float32),
)


def kernel(x_customer, x_recipe, edge_index, edge_label_index,
           W1_l_c2r, b1_l_c2r, W1_r_c2r, W1_l_r2c, b1_l_r2c, W1_r_r2c,
           W2_l_c2r, b2_l_c2r, W2_r_c2r, W2_l_r2c, b2_l_r2c, W2_r_r2c,
           dec_W1, dec_b1, dec_W2, dec_b2):
    src_c = edge_index[0]
    dst_r = edge_index[1]
    row = edge_label_index[0]
    col = edge_label_index[1]

    zeros_nd = jnp.zeros((N, D), jnp.float32)
    zeros_np = jnp.zeros((NPAD,), jnp.float32)
    ones_ech = jnp.ones((ECH,), jnp.float32)

    cat_idx = jnp.concatenate([dst_r, src_c])
    cnts = _counts(cat_idx, ones_ech, zeros_np)
    cnt_rec = cnts[0].reshape(NPAD, 1)
    cnt_cust = cnts[1].reshape(NPAD, 1)

    b1c2r = b1_l_c2r.reshape(1, D)
    b1r2c = b1_l_r2c.reshape(1, D)
    b2c2r = b2_l_c2r.reshape(1, D)
    b2r2c = b2_l_r2c.reshape(1, D)
    dw_top = dec_W1[:D]
    dw_bot = dec_W1[D:]
    eb_cust = dec_b1.reshape(1, D)
    eb_rec = jnp.zeros((1, D), jnp.float32)

    agg1_rec = _segment_sum(x_customer, src_c, dst_r, zeros_nd)
    agg1_cust = _segment_sum(x_recipe, dst_r, src_c, zeros_nd)
    h_rec = _combine_relu(agg1_rec, cnt_rec, x_recipe, W1_l_c2r, b1c2r, W1_r_c2r)
    h_cust = _combine_relu(agg1_cust, cnt_cust, x_customer, W1_l_r2c, b1r2c, W1_r_r2c)

    agg2_rec = _segment_sum(h_cust, src_c, dst_r, zeros_nd)
    agg2_cust = _segment_sum(h_rec, dst_r, src_c, zeros_nd)
    p_rec = _combine_proj(agg2_rec, cnt_rec, h_rec, W2_l_c2r, b2c2r, W2_r_c2r,
                          dw_bot, eb_rec)
    p_cust = _combine_proj(agg2_cust, cnt_cust, h_cust, W2_l_r2c, b2r2c, W2_r_r2c,
                           dw_top, eb_cust)

    g = _gather_pair(p_cust, p_rec, row, col)
    out = _decode(g, dec_W2, dec_b2.reshape(1, 1))
    return out.reshape(-1)


# speedup vs baseline: 6.3568x; 1.5947x over previous
import functools

import jax
import jax.numpy as jnp
from jax import lax
from jax.experimental import pallas as pl
from jax.experimental.pallas import tpu as pltpu
from jax.experimental.pallas import tpu_sc as plsc

NC, NS, LANES = 2, 16, 16
NW = NC * NS
N = 10000
D = 128
E = 320000
NLBL = 65536

EPW = E // NW
ECH = 80
NECH = EPW // ECH

EPT = E // NS
NCCH = EPT // ECH

NPAD = 10240
HPT = NPAD // NS

LPW = NLBL // NW
LCH = 128
NLCH = LPW // LCH

ROWS_PT = 624
ROWS_REM = N - NS * ROWS_PT

_MESH = plsc.VectorSubcoreMesh(core_axis_name="c", subcore_axis_name="s")
_PREC = lax.Precision.HIGHEST


def _dot(a, b):
    return lax.dot_general(a, b, (((1,), (0,)), ((), ())),
                           precision=_PREC, preferred_element_type=jnp.float32)


def _segsum_body(x_hbm, src_hbm, dst_hbm, zeros_hbm, zerosp_hbm, ones_hbm,
                 out_hbm, cnt_hbm, acc_sh, cnt_sh, ones_v, bufs):
    c = lax.axis_index("c")
    s = lax.axis_index("s")
    wid = c * NS + s
    r0 = s * ROWS_PT
    pltpu.sync_copy(zeros_hbm.at[pl.ds(r0, ROWS_PT)], acc_sh.at[pl.ds(r0, ROWS_PT)])
    pltpu.sync_copy(zerosp_hbm.at[pl.ds(s * HPT, HPT)], cnt_sh.at[pl.ds(s * HPT, HPT)])
    pltpu.sync_copy(ones_hbm, ones_v)

    @pl.when(s == 0)
    def _init_rem():
        pltpu.sync_copy(zeros_hbm.at[pl.ds(NS * ROWS_PT, ROWS_REM)],
                        acc_sh.at[pl.ds(NS * ROWS_PT, ROWS_REM)])

    plsc.subcore_barrier()

    base0 = wid * EPW

    def _issue(i, b):
        idx_s, idx_d, rows_v, sem = bufs[b]
        pltpu.sync_copy(src_hbm.at[pl.ds(base0 + i * ECH, ECH)], idx_s)
        pltpu.sync_copy(dst_hbm.at[pl.ds(base0 + i * ECH, ECH)], idx_d)
        pltpu.async_copy(x_hbm.at[idx_s], rows_v, sem)

    def _drain(i, b):
        idx_s, idx_d, rows_v, sem = bufs[b]
        pltpu.make_async_copy(x_hbm.at[idx_s], rows_v, sem).wait()
        pltpu.sync_copy(rows_v, acc_sh.at[idx_d], add=True)
        pltpu.sync_copy(ones_v, cnt_sh.at[idx_d], add=True)

    _issue(0, 0)

    @pl.loop(0, NECH, step=2)
    def _chunk(i):
        @pl.when(i + 1 < NECH)
        def _issue_odd():
            _issue(i + 1, 1)

        _drain(i, 0)

        @pl.when(i + 2 < NECH)
        def _issue_even():
            _issue(i + 2, 0)

        @pl.when(i + 1 < NECH)
        def _drain_odd():
            _drain(i + 1, 1)

    plsc.subcore_barrier()
    pltpu.sync_copy(acc_sh.at[pl.ds(r0, ROWS_PT)],
                    out_hbm.at[c, pl.ds(r0, ROWS_PT)])
    pltpu.sync_copy(cnt_sh.at[pl.ds(s * HPT, HPT)],
                    cnt_hbm.at[c, pl.ds(s * HPT, HPT)])

    @pl.when(s == 0)
    def _export_rem():
        pltpu.sync_copy(acc_sh.at[pl.ds(NS * ROWS_PT, ROWS_REM)],
                        out_hbm.at[c, pl.ds(NS * ROWS_PT, ROWS_REM)])


@functools.partial(
    pl.kernel,
    out_type=(jax.ShapeDtypeStruct((NC, N, D), jnp.float32),
              jax.ShapeDtypeStruct((NC, NPAD), jnp.float32)),
    mesh=_MESH,
    scratch_types=[
        pltpu.VMEM_SHARED((N, D), jnp.float32),
        pltpu.VMEM_SHARED((NPAD,), jnp.float32),
        pltpu.VMEM((ECH,), jnp.float32),
        pltpu.VMEM((ECH,), jnp.int32),
        pltpu.VMEM((ECH,), jnp.int32),
        pltpu.VMEM((ECH, D), jnp.float32),
        pltpu.SemaphoreType.DMA,
        pltpu.VMEM((ECH,), jnp.int32),
        pltpu.VMEM((ECH,), jnp.int32),
        pltpu.VMEM((ECH, D), jnp.float32),
        pltpu.SemaphoreType.DMA,
    ],
)
def _segment_sum(x_hbm, src_hbm, dst_hbm, zeros_hbm, zerosp_hbm, ones_hbm,
                 out_hbm, cnt_hbm, acc_sh, cnt_sh, ones_v,
                 idx_s0, idx_d0, rows0, sem0, idx_s1, idx_d1, rows1, sem1):
    _segsum_body(x_hbm, src_hbm, dst_hbm, zeros_hbm, zerosp_hbm, ones_hbm,
                 out_hbm, cnt_hbm, acc_sh, cnt_sh, ones_v,
                 ((idx_s0, idx_d0, rows0, sem0), (idx_s1, idx_d1, rows1, sem1)))


def _gather_pair_body(pc_hbm, pr_hbm, row_hbm, col_hbm, g_hbm, bufs):
    c = lax.axis_index("c")
    s = lax.axis_index("s")
    wid = c * NS + s
    base0 = wid * LPW

    def _issue(i, b):
        ridx, cidx, buf_v, sem_a, sem_b = bufs[b]
        base = base0 + i * LCH
        pltpu.sync_copy(row_hbm.at[pl.ds(base, LCH)], ridx)
        pltpu.sync_copy(col_hbm.at[pl.ds(base, LCH)], cidx)
        pltpu.async_copy(pc_hbm.at[ridx], buf_v, sem_a)

    def _drain(i, b):
        ridx, cidx, buf_v, sem_a, sem_b = bufs[b]
        pltpu.make_async_copy(pc_hbm.at[ridx], buf_v, sem_a).wait()
        pltpu.async_copy(pr_hbm.at[cidx], buf_v, sem_b, add=True).wait()
        pltpu.sync_copy(buf_v, g_hbm.at[pl.ds(base0 + i * LCH, LCH)])

    _issue(0, 0)

    @pl.loop(0, NLCH, step=2)
    def _chunk(i):
        @pl.when(i + 1 < NLCH)
        def _issue_odd():
            _issue(i + 1, 1)

        _drain(i, 0)

        @pl.when(i + 2 < NLCH)
        def _issue_even():
            _issue(i + 2, 0)

        @pl.when(i + 1 < NLCH)
        def _drain_odd():
            _drain(i + 1, 1)


@functools.partial(
    pl.kernel,
    out_type=jax.ShapeDtypeStruct((NLBL, D), jnp.float32),
    mesh=_MESH,
    scratch_types=[
        pltpu.VMEM((LCH,), jnp.int32),
        pltpu.VMEM((LCH,), jnp.int32),
        pltpu.VMEM((LCH, D), jnp.float32),
        pltpu.SemaphoreType.DMA,
        pltpu.SemaphoreType.DMA,
        pltpu.VMEM((LCH,), jnp.int32),
        pltpu.VMEM((LCH,), jnp.int32),
        pltpu.VMEM((LCH, D), jnp.float32),
        pltpu.SemaphoreType.DMA,
        pltpu.SemaphoreType.DMA,
    ],
)
def _gather_pair(pc_hbm, pr_hbm, row_hbm, col_hbm, g_hbm,
                 r0, c0, b0, sa0, sb0, r1, c1, b1, sa1, sb1):
    _gather_pair_body(pc_hbm, pr_hbm, row_hbm, col_hbm, g_hbm,
                      ((r0, c0, b0, sa0, sb0), (r1, c1, b1, sa1, sb1)))


BM = 2000


def _combine_relu_body(acc_ref, cnt_ref, x_ref, wl_ref, bl_ref, wr_ref, o_ref):
    inv = 1.0 / jnp.maximum(cnt_ref[0] + cnt_ref[1], 1.0)
    mean = (acc_ref[0] + acc_ref[1]) * inv
    h = _dot(mean, wl_ref[...]) + _dot(x_ref[...], wr_ref[...]) + bl_ref[...]
    o_ref[...] = jnp.maximum(h, 0.0)


def _combine_proj_body(acc_ref, cnt_ref, x_ref, wl_ref, bl_ref, wr_ref,
                       dw_ref, eb_ref, o_ref):
    inv = 1.0 / jnp.maximum(cnt_ref[0] + cnt_ref[1], 1.0)
    mean = (acc_ref[0] + acc_ref[1]) * inv
    dw = dw_ref[...]
    wld = _dot(wl_ref[...], dw)
    wrd = _dot(wr_ref[...], dw)
    bld = _dot(bl_ref[...], dw)
    o_ref[...] = _dot(mean, wld) + _dot(x_ref[...], wrd) + bld + eb_ref[...]


_acc_spec = pl.BlockSpec((NC, BM, D), lambda i: (0, i, 0))
_cnt_spec = pl.BlockSpec((NC, BM, 1), lambda i: (0, i, 0))
_x_spec = pl.BlockSpec((BM, D), lambda i: (i, 0))
_w_spec = pl.BlockSpec((D, D), lambda i: (0, 0))
_b_spec = pl.BlockSpec((1, D), lambda i: (0, 0))

_combine_relu = pl.pallas_call(
    _combine_relu_body,
    grid=(N // BM,),
    in_specs=[_acc_spec, _cnt_spec, _x_spec, _w_spec, _b_spec, _w_spec],
    out_specs=_x_spec,
    out_shape=jax.ShapeDtypeStruct((N, D), jnp.float32),
)

_combine_proj = pl.pallas_call(
    _combine_proj_body,
    grid=(N // BM,),
    in_specs=[_acc_spec, _cnt_spec, _x_spec, _w_spec, _b_spec, _w_spec,
              _w_spec, _b_spec],
    out_specs=_x_spec,
    out_shape=jax.ShapeDtypeStruct((N, D), jnp.float32),
)


def _decode_body(g_ref, w2_ref, b2_ref, o_ref):
    o_ref[...] = _dot(jnp.maximum(g_ref[...], 0.0), w2_ref[...]) + b2_ref[...]


BL = 8192

_decode = pl.pallas_call(
    _decode_body,
    grid=(NLBL // BL,),
    in_specs=[pl.BlockSpec((BL, D), lambda i: (i, 0)),
              pl.BlockSpec((D, 1), lambda i: (0, 0)),
              pl.BlockSpec((1, 1), lambda i: (0, 0))],
    out_specs=pl.BlockSpec((BL, 1), lambda i: (i, 0)),
    out_shape=jax.ShapeDtypeStruct((NLBL, 1), jnp.float32),
)


def kernel(x_customer, x_recipe, edge_index, edge_label_index,
           W1_l_c2r, b1_l_c2r, W1_r_c2r, W1_l_r2c, b1_l_r2c, W1_r_r2c,
           W2_l_c2r, b2_l_c2r, W2_r_c2r, W2_l_r2c, b2_l_r2c, W2_r_r2c,
           dec_W1, dec_b1, dec_W2, dec_b2):
    src_c = edge_index[0]
    dst_r = edge_index[1]
    row = edge_label_index[0]
    col = edge_label_index[1]

    zeros_nd = jnp.zeros((N, D), jnp.float32)
    zeros_np = jnp.zeros((NPAD,), jnp.float32)
    ones_ech = jnp.ones((ECH,), jnp.float32)

    b1c2r = b1_l_c2r.reshape(1, D)
    b1r2c = b1_l_r2c.reshape(1, D)
    b2c2r = b2_l_c2r.reshape(1, D)
    b2r2c = b2_l_r2c.reshape(1, D)
    dw_top = dec_W1[:D]
    dw_bot = dec_W1[D:]
    eb_cust = dec_b1.reshape(1, D)
    eb_rec = jnp.zeros((1, D), jnp.float32)

    agg1_rec, cntp_rec = _segment_sum(x_customer, src_c, dst_r,
                                      zeros_nd, zeros_np, ones_ech)
    agg1_cust, cntp_cust = _segment_sum(x_recipe, dst_r, src_c,
                                        zeros_nd, zeros_np, ones_ech)
    cnt_rec = cntp_rec.reshape(NC, NPAD, 1)
    cnt_cust = cntp_cust.reshape(NC, NPAD, 1)
    h_rec = _combine_relu(agg1_rec, cnt_rec, x_recipe, W1_l_c2r, b1c2r, W1_r_c2r)
    h_cust = _combine_relu(agg1_cust, cnt_cust, x_customer, W1_l_r2c, b1r2c, W1_r_r2c)

    agg2_rec, cntp_rec2 = _segment_sum(h_cust, src_c, dst_r,
                                       zeros_nd, zeros_np, ones_ech)
    agg2_cust, cntp_cust2 = _segment_sum(h_rec, dst_r, src_c,
                                         zeros_nd, zeros_np, ones_ech)
    p_rec = _combine_proj(agg2_rec, cntp_rec2.reshape(NC, NPAD, 1), h_rec,
                          W2_l_c2r, b2c2r, W2_r_c2r, dw_bot, eb_rec)
    p_cust = _combine_proj(agg2_cust, cntp_cust2.reshape(NC, NPAD, 1), h_cust,
                           W2_l_r2c, b2r2c, W2_r_r2c, dw_top, eb_cust)

    g = _gather_pair(p_cust, p_rec, row, col)
    out = _decode(g, dec_W2, dec_b2.reshape(1, 1))
    return out.reshape(-1)


# speedup vs baseline: 7.4674x; 1.1747x over previous
import functools

import jax
import jax.numpy as jnp
from jax import lax
from jax.experimental import pallas as pl
from jax.experimental.pallas import tpu as pltpu
from jax.experimental.pallas import tpu_sc as plsc

NC, NS, LANES = 2, 16, 16
NW = NC * NS
N = 10000
D = 128
E = 320000
NLBL = 65536

EPW = E // NW
ECH = 128
NECHF = 78
ECHT = EPW - NECHF * ECH

NPAD = 10240
HPT = NPAD // NS

LPW = NLBL // NW
LCH = 128
NLCH = LPW // LCH

ROWS_PT = 624
ROWS_REM = N - NS * ROWS_PT

_MESH = plsc.VectorSubcoreMesh(core_axis_name="c", subcore_axis_name="s")
_PREC = lax.Precision.HIGHEST


def _dot(a, b):
    return lax.dot_general(a, b, (((1,), (0,)), ((), ())),
                           precision=_PREC, preferred_element_type=jnp.float32)


def _segsum_body(x_hbm, src_hbm, dst_hbm, zeros_hbm, zerosp_hbm, ones_hbm,
                 out_hbm, cnt_hbm, acc_sh, cnt_sh, ones_v, bufs, tail):
    c = lax.axis_index("c")
    s = lax.axis_index("s")
    wid = c * NS + s
    r0 = s * ROWS_PT
    pltpu.sync_copy(zeros_hbm.at[pl.ds(r0, ROWS_PT)], acc_sh.at[pl.ds(r0, ROWS_PT)])
    pltpu.sync_copy(zerosp_hbm.at[pl.ds(s * HPT, HPT)], cnt_sh.at[pl.ds(s * HPT, HPT)])
    pltpu.sync_copy(ones_hbm, ones_v)

    @pl.when(s == 0)
    def _init_rem():
        pltpu.sync_copy(zeros_hbm.at[pl.ds(NS * ROWS_PT, ROWS_REM)],
                        acc_sh.at[pl.ds(NS * ROWS_PT, ROWS_REM)])

    plsc.subcore_barrier()

    base0 = wid * EPW

    def _issue(i, b):
        idx_s, idx_d, rows_v, sem = bufs[b]
        pltpu.sync_copy(src_hbm.at[pl.ds(base0 + i * ECH, ECH)], idx_s)
        pltpu.sync_copy(dst_hbm.at[pl.ds(base0 + i * ECH, ECH)], idx_d)
        pltpu.async_copy(x_hbm.at[idx_s], rows_v, sem)

    def _drain(i, b):
        idx_s, idx_d, rows_v, sem = bufs[b]
        pltpu.make_async_copy(x_hbm.at[idx_s], rows_v, sem).wait()
        pltpu.sync_copy(rows_v, acc_sh.at[idx_d], add=True)
        pltpu.sync_copy(ones_v, cnt_sh.at[idx_d], add=True)

    _issue(0, 0)

    @pl.loop(0, NECHF, step=2)
    def _chunk(i):
        _issue(i + 1, 1)
        _drain(i, 0)

        @pl.when(i + 2 < NECHF)
        def _issue_even():
            _issue(i + 2, 0)

        _drain(i + 1, 1)

    idx_st, idx_dt, rows_t, ones_t, sem_t = tail
    base_t = base0 + NECHF * ECH
    pltpu.sync_copy(src_hbm.at[pl.ds(base_t, ECHT)], idx_st)
    pltpu.sync_copy(dst_hbm.at[pl.ds(base_t, ECHT)], idx_dt)
    pltpu.sync_copy(ones_hbm.at[pl.ds(0, ECHT)], ones_t)
    pltpu.async_copy(x_hbm.at[idx_st], rows_t, sem_t).wait()
    pltpu.sync_copy(rows_t, acc_sh.at[idx_dt], add=True)
    pltpu.sync_copy(ones_t, cnt_sh.at[idx_dt], add=True)

    plsc.subcore_barrier()
    pltpu.sync_copy(acc_sh.at[pl.ds(r0, ROWS_PT)],
                    out_hbm.at[c, pl.ds(r0, ROWS_PT)])
    pltpu.sync_copy(cnt_sh.at[pl.ds(s * HPT, HPT)],
                    cnt_hbm.at[c, pl.ds(s * HPT, HPT)])

    @pl.when(s == 0)
    def _export_rem():
        pltpu.sync_copy(acc_sh.at[pl.ds(NS * ROWS_PT, ROWS_REM)],
                        out_hbm.at[c, pl.ds(NS * ROWS_PT, ROWS_REM)])


@functools.partial(
    pl.kernel,
    out_type=(jax.ShapeDtypeStruct((NC, N, D), jnp.float32),
              jax.ShapeDtypeStruct((NC, NPAD), jnp.float32)),
    mesh=_MESH,
    scratch_types=[
        pltpu.VMEM_SHARED((N, D), jnp.float32),
        pltpu.VMEM_SHARED((NPAD,), jnp.float32),
        pltpu.VMEM((ECH,), jnp.float32),
        pltpu.VMEM((ECH,), jnp.int32),
        pltpu.VMEM((ECH,), jnp.int32),
        pltpu.VMEM((ECH, D), jnp.float32),
        pltpu.SemaphoreType.DMA,
        pltpu.VMEM((ECH,), jnp.int32),
        pltpu.VMEM((ECH,), jnp.int32),
        pltpu.VMEM((ECH, D), jnp.float32),
        pltpu.SemaphoreType.DMA,
        pltpu.VMEM((ECHT,), jnp.int32),
        pltpu.VMEM((ECHT,), jnp.int32),
        pltpu.VMEM((ECHT, D), jnp.float32),
        pltpu.VMEM((ECHT,), jnp.float32),
        pltpu.SemaphoreType.DMA,
    ],
)
def _segment_sum(x_hbm, src_hbm, dst_hbm, zeros_hbm, zerosp_hbm, ones_hbm,
                 out_hbm, cnt_hbm, acc_sh, cnt_sh, ones_v,
                 idx_s0, idx_d0, rows0, sem0, idx_s1, idx_d1, rows1, sem1,
                 idx_st, idx_dt, rows_t, ones_t, sem_t):
    _segsum_body(x_hbm, src_hbm, dst_hbm, zeros_hbm, zerosp_hbm, ones_hbm,
                 out_hbm, cnt_hbm, acc_sh, cnt_sh, ones_v,
                 ((idx_s0, idx_d0, rows0, sem0), (idx_s1, idx_d1, rows1, sem1)),
                 (idx_st, idx_dt, rows_t, ones_t, sem_t))


def _gather_pair_body(pc_hbm, pr_hbm, row_hbm, col_hbm, g_hbm, bufs):
    c = lax.axis_index("c")
    s = lax.axis_index("s")
    wid = c * NS + s
    base0 = wid * LPW

    def _issue(i, b):
        ridx, cidx, buf_v, sem_a, sem_b = bufs[b]
        base = base0 + i * LCH
        pltpu.sync_copy(row_hbm.at[pl.ds(base, LCH)], ridx)
        pltpu.sync_copy(col_hbm.at[pl.ds(base, LCH)], cidx)
        pltpu.async_copy(pc_hbm.at[ridx], buf_v, sem_a)

    def _drain(i, b):
        ridx, cidx, buf_v, sem_a, sem_b = bufs[b]
        pltpu.make_async_copy(pc_hbm.at[ridx], buf_v, sem_a).wait()
        pltpu.async_copy(pr_hbm.at[cidx], buf_v, sem_b, add=True).wait()
        pltpu.sync_copy(buf_v, g_hbm.at[pl.ds(base0 + i * LCH, LCH)])

    _issue(0, 0)

    @pl.loop(0, NLCH, step=2)
    def _chunk(i):
        @pl.when(i + 1 < NLCH)
        def _issue_odd():
            _issue(i + 1, 1)

        _drain(i, 0)

        @pl.when(i + 2 < NLCH)
        def _issue_even():
            _issue(i + 2, 0)

        @pl.when(i + 1 < NLCH)
        def _drain_odd():
            _drain(i + 1, 1)


@functools.partial(
    pl.kernel,
    out_type=jax.ShapeDtypeStruct((NLBL, D), jnp.float32),
    mesh=_MESH,
    scratch_types=[
        pltpu.VMEM((LCH,), jnp.int32),
        pltpu.VMEM((LCH,), jnp.int32),
        pltpu.VMEM((LCH, D), jnp.float32),
        pltpu.SemaphoreType.DMA,
        pltpu.SemaphoreType.DMA,
        pltpu.VMEM((LCH,), jnp.int32),
        pltpu.VMEM((LCH,), jnp.int32),
        pltpu.VMEM((LCH, D), jnp.float32),
        pltpu.SemaphoreType.DMA,
        pltpu.SemaphoreType.DMA,
    ],
)
def _gather_pair(pc_hbm, pr_hbm, row_hbm, col_hbm, g_hbm,
                 r0, c0, b0, sa0, sb0, r1, c1, b1, sa1, sb1):
    _gather_pair_body(pc_hbm, pr_hbm, row_hbm, col_hbm, g_hbm,
                      ((r0, c0, b0, sa0, sb0), (r1, c1, b1, sa1, sb1)))


BM = 2000


def _combine_relu_body(acc_ref, cnt_ref, x_ref, wl_ref, bl_ref, wr_ref, o_ref):
    inv = 1.0 / jnp.maximum(cnt_ref[0] + cnt_ref[1], 1.0)
    mean = (acc_ref[0] + acc_ref[1]) * inv
    h = _dot(mean, wl_ref[...]) + _dot(x_ref[...], wr_ref[...]) + bl_ref[...]
    o_ref[...] = jnp.maximum(h, 0.0)


def _combine_proj_body(acc_ref, cnt_ref, x_ref, wl_ref, bl_ref, wr_ref,
                       dw_ref, eb_ref, o_ref):
    inv = 1.0 / jnp.maximum(cnt_ref[0] + cnt_ref[1], 1.0)
    mean = (acc_ref[0] + acc_ref[1]) * inv
    dw = dw_ref[...]
    wld = _dot(wl_ref[...], dw)
    wrd = _dot(wr_ref[...], dw)
    bld = _dot(bl_ref[...], dw)
    o_ref[...] = _dot(mean, wld) + _dot(x_ref[...], wrd) + bld + eb_ref[...]


_acc_spec = pl.BlockSpec((NC, BM, D), lambda i: (0, i, 0))
_cnt_spec = pl.BlockSpec((NC, BM, 1), lambda i: (0, i, 0))
_x_spec = pl.BlockSpec((BM, D), lambda i: (i, 0))
_w_spec = pl.BlockSpec((D, D), lambda i: (0, 0))
_b_spec = pl.BlockSpec((1, D), lambda i: (0, 0))

_combine_relu = pl.pallas_call(
    _combine_relu_body,
    grid=(N // BM,),
    in_specs=[_acc_spec, _cnt_spec, _x_spec, _w_spec, _b_spec, _w_spec],
    out_specs=_x_spec,
    out_shape=jax.ShapeDtypeStruct((N, D), jnp.float32),
)

_combine_proj = pl.pallas_call(
    _combine_proj_body,
    grid=(N // BM,),
    in_specs=[_acc_spec, _cnt_spec, _x_spec, _w_spec, _b_spec, _w_spec,
              _w_spec, _b_spec],
    out_specs=_x_spec,
    out_shape=jax.ShapeDtypeStruct((N, D), jnp.float32),
)


def _decode_body(g_ref, w2_ref, b2_ref, o_ref):
    o_ref[...] = _dot(jnp.maximum(g_ref[...], 0.0), w2_ref[...]) + b2_ref[...]


BL = 8192

_decode = pl.pallas_call(
    _decode_body,
    grid=(NLBL // BL,),
    in_specs=[pl.BlockSpec((BL, D), lambda i: (i, 0)),
              pl.BlockSpec((D, 1), lambda i: (0, 0)),
              pl.BlockSpec((1, 1), lambda i: (0, 0))],
    out_specs=pl.BlockSpec((BL, 1), lambda i: (i, 0)),
    out_shape=jax.ShapeDtypeStruct((NLBL, 1), jnp.float32),
)


def kernel(x_customer, x_recipe, edge_index, edge_label_index,
           W1_l_c2r, b1_l_c2r, W1_r_c2r, W1_l_r2c, b1_l_r2c, W1_r_r2c,
           W2_l_c2r, b2_l_c2r, W2_r_c2r, W2_l_r2c, b2_l_r2c, W2_r_r2c,
           dec_W1, dec_b1, dec_W2, dec_b2):
    src_c = edge_index[0]
    dst_r = edge_index[1]
    row = edge_label_index[0]
    col = edge_label_index[1]

    zeros_nd = jnp.zeros((N, D), jnp.float32)
    zeros_np = jnp.zeros((NPAD,), jnp.float32)
    ones_ech = jnp.ones((ECH,), jnp.float32)

    b1c2r = b1_l_c2r.reshape(1, D)
    b1r2c = b1_l_r2c.reshape(1, D)
    b2c2r = b2_l_c2r.reshape(1, D)
    b2r2c = b2_l_r2c.reshape(1, D)
    dw_top = dec_W1[:D]
    dw_bot = dec_W1[D:]
    eb_cust = dec_b1.reshape(1, D)
    eb_rec = jnp.zeros((1, D), jnp.float32)

    agg1_rec, cntp_rec = _segment_sum(x_customer, src_c, dst_r,
                                      zeros_nd, zeros_np, ones_ech)
    agg1_cust, cntp_cust = _segment_sum(x_recipe, dst_r, src_c,
                                        zeros_nd, zeros_np, ones_ech)
    cnt_rec = cntp_rec.reshape(NC, NPAD, 1)
    cnt_cust = cntp_cust.reshape(NC, NPAD, 1)
    h_rec = _combine_relu(agg1_rec, cnt_rec, x_recipe, W1_l_c2r, b1c2r, W1_r_c2r)
    h_cust = _combine_relu(agg1_cust, cnt_cust, x_customer, W1_l_r2c, b1r2c, W1_r_r2c)

    agg2_rec, cntp_rec2 = _segment_sum(h_cust, src_c, dst_r,
                                       zeros_nd, zeros_np, ones_ech)
    agg2_cust, cntp_cust2 = _segment_sum(h_rec, dst_r, src_c,
                                         zeros_nd, zeros_np, ones_ech)
    p_rec = _combine_proj(agg2_rec, cntp_rec2.reshape(NC, NPAD, 1), h_rec,
                          W2_l_c2r, b2c2r, W2_r_c2r, dw_bot, eb_rec)
    p_cust = _combine_proj(agg2_cust, cntp_cust2.reshape(NC, NPAD, 1), h_cust,
                           W2_l_r2c, b2r2c, W2_r_r2c, dw_top, eb_cust)

    g = _gather_pair(p_cust, p_rec, row, col)
    out = _decode(g, dec_W2, dec_b2.reshape(1, 1))
    return out.reshape(-1)


# speedup vs baseline: 8.4126x; 1.1266x over previous
import functools

import jax
import jax.numpy as jnp
from jax import lax
from jax.experimental import pallas as pl
from jax.experimental.pallas import tpu as pltpu
from jax.experimental.pallas import tpu_sc as plsc

NC, NS, LANES = 2, 16, 16
NW = NC * NS
N = 10000
D = 128
E = 320000
NLBL = 65536

EPW = E // NW
ECH = 112
NECHF = 89
NECHL = 87
ECHT = EPW - NECHF * ECH

NPAD = 10240
HPT = NPAD // NS

LPW = NLBL // NW
LCH = 128
NLCH = LPW // LCH

ROWS_PT = 624
ROWS_REM = N - NS * ROWS_PT

_MESH = plsc.VectorSubcoreMesh(core_axis_name="c", subcore_axis_name="s")
_PREC = lax.Precision.HIGHEST


def _dot(a, b):
    return lax.dot_general(a, b, (((1,), (0,)), ((), ())),
                           precision=_PREC, preferred_element_type=jnp.float32)


def _segsum_body(x_hbm, src_hbm, dst_hbm, zeros_hbm, zerosp_hbm, ones_hbm,
                 out_hbm, cnt_hbm, acc_sh, cnt_sh, ones_v, bufs, tail):
    c = lax.axis_index("c")
    s = lax.axis_index("s")
    wid = c * NS + s
    r0 = s * ROWS_PT
    pltpu.sync_copy(zeros_hbm.at[pl.ds(r0, ROWS_PT)], acc_sh.at[pl.ds(r0, ROWS_PT)])
    pltpu.sync_copy(zerosp_hbm.at[pl.ds(s * HPT, HPT)], cnt_sh.at[pl.ds(s * HPT, HPT)])
    pltpu.sync_copy(ones_hbm, ones_v)

    @pl.when(s == 0)
    def _init_rem():
        pltpu.sync_copy(zeros_hbm.at[pl.ds(NS * ROWS_PT, ROWS_REM)],
                        acc_sh.at[pl.ds(NS * ROWS_PT, ROWS_REM)])

    plsc.subcore_barrier()

    base0 = wid * EPW

    def _issue(k, b):
        idx_s, idx_d, rows_v, gsem, ssem = bufs[b]
        pltpu.sync_copy(src_hbm.at[pl.ds(base0 + k * ECH, ECH)], idx_s)
        pltpu.sync_copy(dst_hbm.at[pl.ds(base0 + k * ECH, ECH)], idx_d)
        pltpu.async_copy(x_hbm.at[idx_s], rows_v, gsem)

    def _wait_scatter(b):
        idx_s, idx_d, rows_v, gsem, ssem = bufs[b]
        pltpu.make_async_copy(rows_v, acc_sh.at[idx_d], ssem).wait()

    def _process(k, b):
        idx_s, idx_d, rows_v, gsem, ssem = bufs[b]
        pltpu.make_async_copy(x_hbm.at[idx_s], rows_v, gsem).wait()
        pltpu.async_copy(rows_v, acc_sh.at[idx_d], ssem, add=True)
        pltpu.sync_copy(ones_v, cnt_sh.at[idx_d], add=True)

    _issue(0, 0)
    _issue(1, 1)

    @pl.loop(0, NECHL, step=3)
    def _chunk(i):
        for j in range(3):
            k = i + j
            _process(k, j)
            b2 = (j + 2) % 3

            @pl.when(k >= 1)
            def _reuse_wait():
                _wait_scatter(b2)

            _issue(k + 2, b2)

    _process(NECHL, 0)
    _process(NECHL + 1, 1)
    _wait_scatter(2)
    idx_st, idx_dt, sem_t = tail
    rows2 = bufs[2][2]
    base_t = base0 + NECHF * ECH
    pltpu.sync_copy(src_hbm.at[pl.ds(base_t, ECHT)], idx_st)
    pltpu.sync_copy(dst_hbm.at[pl.ds(base_t, ECHT)], idx_dt)
    pltpu.async_copy(x_hbm.at[idx_st], rows2.at[pl.ds(0, ECHT)], sem_t).wait()
    pltpu.sync_copy(rows2.at[pl.ds(0, ECHT)], acc_sh.at[idx_dt], add=True)
    pltpu.sync_copy(ones_v.at[pl.ds(0, ECHT)], cnt_sh.at[idx_dt], add=True)
    _wait_scatter(0)
    _wait_scatter(1)

    plsc.subcore_barrier()
    pltpu.sync_copy(acc_sh.at[pl.ds(r0, ROWS_PT)],
                    out_hbm.at[c, pl.ds(r0, ROWS_PT)])
    pltpu.sync_copy(cnt_sh.at[pl.ds(s * HPT, HPT)],
                    cnt_hbm.at[c, pl.ds(s * HPT, HPT)])

    @pl.when(s == 0)
    def _export_rem():
        pltpu.sync_copy(acc_sh.at[pl.ds(NS * ROWS_PT, ROWS_REM)],
                        out_hbm.at[c, pl.ds(NS * ROWS_PT, ROWS_REM)])


@functools.partial(
    pl.kernel,
    out_type=(jax.ShapeDtypeStruct((NC, N, D), jnp.float32),
              jax.ShapeDtypeStruct((NC, NPAD), jnp.float32)),
    mesh=_MESH,
    scratch_types=[
        pltpu.VMEM_SHARED((N, D), jnp.float32),
        pltpu.VMEM_SHARED((NPAD,), jnp.float32),
        pltpu.VMEM((ECH,), jnp.float32),
    ] + [
        t
        for _ in range(3)
        for t in (pltpu.VMEM((ECH,), jnp.int32),
                  pltpu.VMEM((ECH,), jnp.int32),
                  pltpu.VMEM((ECH, D), jnp.float32),
                  pltpu.SemaphoreType.DMA,
                  pltpu.SemaphoreType.DMA)
    ] + [
        pltpu.VMEM((ECHT,), jnp.int32),
        pltpu.VMEM((ECHT,), jnp.int32),
        pltpu.SemaphoreType.DMA,
    ],
)
def _segment_sum(x_hbm, src_hbm, dst_hbm, zeros_hbm, zerosp_hbm, ones_hbm,
                 out_hbm, cnt_hbm, acc_sh, cnt_sh, ones_v, *rest):
    bufs = tuple(rest[5 * b:5 * b + 5] for b in range(3))
    _segsum_body(x_hbm, src_hbm, dst_hbm, zeros_hbm, zerosp_hbm, ones_hbm,
                 out_hbm, cnt_hbm, acc_sh, cnt_sh, ones_v, bufs, rest[15:18])


def _gather_pair_body(pc_hbm, pr_hbm, row_hbm, col_hbm, g_hbm, bufs):
    c = lax.axis_index("c")
    s = lax.axis_index("s")
    wid = c * NS + s
    base0 = wid * LPW

    def _issue(i, b):
        ridx, cidx, buf_v, sem_a, sem_b = bufs[b]
        base = base0 + i * LCH
        pltpu.sync_copy(row_hbm.at[pl.ds(base, LCH)], ridx)
        pltpu.sync_copy(col_hbm.at[pl.ds(base, LCH)], cidx)
        pltpu.async_copy(pc_hbm.at[ridx], buf_v, sem_a)

    def _drain(i, b):
        ridx, cidx, buf_v, sem_a, sem_b = bufs[b]
        pltpu.make_async_copy(pc_hbm.at[ridx], buf_v, sem_a).wait()
        pltpu.async_copy(pr_hbm.at[cidx], buf_v, sem_b, add=True).wait()
        pltpu.sync_copy(buf_v, g_hbm.at[pl.ds(base0 + i * LCH, LCH)])

    _issue(0, 0)

    @pl.loop(0, NLCH, step=2)
    def _chunk(i):
        @pl.when(i + 1 < NLCH)
        def _issue_odd():
            _issue(i + 1, 1)

        _drain(i, 0)

        @pl.when(i + 2 < NLCH)
        def _issue_even():
            _issue(i + 2, 0)

        @pl.when(i + 1 < NLCH)
        def _drain_odd():
            _drain(i + 1, 1)


@functools.partial(
    pl.kernel,
    out_type=jax.ShapeDtypeStruct((NLBL, D), jnp.float32),
    mesh=_MESH,
    scratch_types=[
        pltpu.VMEM((LCH,), jnp.int32),
        pltpu.VMEM((LCH,), jnp.int32),
        pltpu.VMEM((LCH, D), jnp.float32),
        pltpu.SemaphoreType.DMA,
        pltpu.SemaphoreType.DMA,
        pltpu.VMEM((LCH,), jnp.int32),
        pltpu.VMEM((LCH,), jnp.int32),
        pltpu.VMEM((LCH, D), jnp.float32),
        pltpu.SemaphoreType.DMA,
        pltpu.SemaphoreType.DMA,
    ],
)
def _gather_pair(pc_hbm, pr_hbm, row_hbm, col_hbm, g_hbm,
                 r0, c0, b0, sa0, sb0, r1, c1, b1, sa1, sb1):
    _gather_pair_body(pc_hbm, pr_hbm, row_hbm, col_hbm, g_hbm,
                      ((r0, c0, b0, sa0, sb0), (r1, c1, b1, sa1, sb1)))


BM = 2000


def _combine_relu_body(acc_ref, cnt_ref, x_ref, wl_ref, bl_ref, wr_ref, o_ref):
    inv = 1.0 / jnp.maximum(cnt_ref[0] + cnt_ref[1], 1.0)
    mean = (acc_ref[0] + acc_ref[1]) * inv
    h = _dot(mean, wl_ref[...]) + _dot(x_ref[...], wr_ref[...]) + bl_ref[...]
    o_ref[...] = jnp.maximum(h, 0.0)


def _combine_proj_body(acc_ref, cnt_ref, x_ref, wl_ref, bl_ref, wr_ref,
                       dw_ref, eb_ref, o_ref):
    inv = 1.0 / jnp.maximum(cnt_ref[0] + cnt_ref[1], 1.0)
    mean = (acc_ref[0] + acc_ref[1]) * inv
    dw = dw_ref[...]
    wld = _dot(wl_ref[...], dw)
    wrd = _dot(wr_ref[...], dw)
    bld = _dot(bl_ref[...], dw)
    o_ref[...] = _dot(mean, wld) + _dot(x_ref[...], wrd) + bld + eb_ref[...]


_acc_spec = pl.BlockSpec((NC, BM, D), lambda i: (0, i, 0))
_cnt_spec = pl.BlockSpec((NC, BM, 1), lambda i: (0, i, 0))
_x_spec = pl.BlockSpec((BM, D), lambda i: (i, 0))
_w_spec = pl.BlockSpec((D, D), lambda i: (0, 0))
_b_spec = pl.BlockSpec((1, D), lambda i: (0, 0))

_combine_relu = pl.pallas_call(
    _combine_relu_body,
    grid=(N // BM,),
    in_specs=[_acc_spec, _cnt_spec, _x_spec, _w_spec, _b_spec, _w_spec],
    out_specs=_x_spec,
    out_shape=jax.ShapeDtypeStruct((N, D), jnp.float32),
)

_combine_proj = pl.pallas_call(
    _combine_proj_body,
    grid=(N // BM,),
    in_specs=[_acc_spec, _cnt_spec, _x_spec, _w_spec, _b_spec, _w_spec,
              _w_spec, _b_spec],
    out_specs=_x_spec,
    out_shape=jax.ShapeDtypeStruct((N, D), jnp.float32),
)


def _decode_body(g_ref, w2_ref, b2_ref, o_ref):
    o_ref[...] = _dot(jnp.maximum(g_ref[...], 0.0), w2_ref[...]) + b2_ref[...]


BL = 8192

_decode = pl.pallas_call(
    _decode_body,
    grid=(NLBL // BL,),
    in_specs=[pl.BlockSpec((BL, D), lambda i: (i, 0)),
              pl.BlockSpec((D, 1), lambda i: (0, 0)),
              pl.BlockSpec((1, 1), lambda i: (0, 0))],
    out_specs=pl.BlockSpec((BL, 1), lambda i: (i, 0)),
    out_shape=jax.ShapeDtypeStruct((NLBL, 1), jnp.float32),
)


def kernel(x_customer, x_recipe, edge_index, edge_label_index,
           W1_l_c2r, b1_l_c2r, W1_r_c2r, W1_l_r2c, b1_l_r2c, W1_r_r2c,
           W2_l_c2r, b2_l_c2r, W2_r_c2r, W2_l_r2c, b2_l_r2c, W2_r_r2c,
           dec_W1, dec_b1, dec_W2, dec_b2):
    src_c = edge_index[0]
    dst_r = edge_index[1]
    row = edge_label_index[0]
    col = edge_label_index[1]

    zeros_nd = jnp.zeros((N, D), jnp.float32)
    zeros_np = jnp.zeros((NPAD,), jnp.float32)
    ones_ech = jnp.ones((ECH,), jnp.float32)

    b1c2r = b1_l_c2r.reshape(1, D)
    b1r2c = b1_l_r2c.reshape(1, D)
    b2c2r = b2_l_c2r.reshape(1, D)
    b2r2c = b2_l_r2c.reshape(1, D)
    dw_top = dec_W1[:D]
    dw_bot = dec_W1[D:]
    eb_cust = dec_b1.reshape(1, D)
    eb_rec = jnp.zeros((1, D), jnp.float32)

    agg1_rec, cntp_rec = _segment_sum(x_customer, src_c, dst_r,
                                      zeros_nd, zeros_np, ones_ech)
    agg1_cust, cntp_cust = _segment_sum(x_recipe, dst_r, src_c,
                                        zeros_nd, zeros_np, ones_ech)
    cnt_rec = cntp_rec.reshape(NC, NPAD, 1)
    cnt_cust = cntp_cust.reshape(NC, NPAD, 1)
    h_rec = _combine_relu(agg1_rec, cnt_rec, x_recipe, W1_l_c2r, b1c2r, W1_r_c2r)
    h_cust = _combine_relu(agg1_cust, cnt_cust, x_customer, W1_l_r2c, b1r2c, W1_r_r2c)

    agg2_rec, cntp_rec2 = _segment_sum(h_cust, src_c, dst_r,
                                       zeros_nd, zeros_np, ones_ech)
    agg2_cust, cntp_cust2 = _segment_sum(h_rec, dst_r, src_c,
                                         zeros_nd, zeros_np, ones_ech)
    p_rec = _combine_proj(agg2_rec, cntp_rec2.reshape(NC, NPAD, 1), h_rec,
                          W2_l_c2r, b2c2r, W2_r_c2r, dw_bot, eb_rec)
    p_cust = _combine_proj(agg2_cust, cntp_cust2.reshape(NC, NPAD, 1), h_cust,
                           W2_l_r2c, b2r2c, W2_r_r2c, dw_top, eb_cust)

    g = _gather_pair(p_cust, p_rec, row, col)
    out = _decode(g, dec_W2, dec_b2.reshape(1, 1))
    return out.reshape(-1)
